# Initial kernel scaffold; baseline (speedup 1.0000x reference)
import jax
import jax.numpy as jnp
from jax.experimental import pallas as pl


def kernel(x, embed):
    raise NotImplementedError("write your pallas kernel here")


# speedup vs baseline: 1.2304x; 1.2304x over previous
import jax
import jax.numpy as jnp
from jax.experimental import pallas as pl

DIM = 256
N_CODES = 8192
CHUNK = 2048
BR = 512


def _assign_kernel(x_ref, e2_ref, en_ref, out_ref):
    xb = x_ref[...]
    x_norm = jnp.sum(xb * xb, axis=1, keepdims=True)
    mm2 = jax.lax.dot_general(
        xb, e2_ref[...],
        dimension_numbers=(((1,), (0,)), ((), ())),
        preferred_element_type=jnp.float32,
    )
    t = (x_norm - mm2) + en_ref[...]

    best_idx = None
    run = None
    for c in range(N_CODES // CHUNK):
        tc = t[:, c * CHUNK:(c + 1) * CHUNK]
        m = jnp.min(tc, axis=1, keepdims=True)
        ids = jax.lax.broadcasted_iota(jnp.int32, tc.shape, 1) + c * CHUNK
        idx = jnp.min(jnp.where(tc == m, ids, N_CODES), axis=1)
        mval = m[:, 0]
        mval_b = mval.astype(jnp.bfloat16).astype(jnp.float32)
        if c == 0:
            best_idx = idx
            run = mval_b
        else:
            better = mval < run
            best_idx = jnp.where(better, idx, best_idx)
            run = jnp.where(better, mval_b, run)
    out_ref[0, 0, :] = best_idx


@jax.jit
def kernel(x, embed):
    flat = x.reshape(-1, DIM)
    rows = flat.shape[0]
    nr = rows // BR
    embed2 = embed + embed
    embed_norm = jnp.sum(embed * embed, axis=0, keepdims=True)
    out = pl.pallas_call(
        _assign_kernel,
        grid=(nr,),
        in_specs=[
            pl.BlockSpec((BR, DIM), lambda i: (i, 0)),
            pl.BlockSpec((DIM, N_CODES), lambda i: (0, 0)),
            pl.BlockSpec((1, N_CODES), lambda i: (0, 0)),
        ],
        out_specs=pl.BlockSpec((1, 1, BR), lambda i: (i, 0, 0)),
        out_shape=jax.ShapeDtypeStruct((nr, 1, BR), jnp.int32),
    )(flat, embed2, embed_norm)
    return out.reshape(x.shape[:-1])


# speedup vs baseline: 1.4059x; 1.1427x over previous
import jax
import jax.numpy as jnp
from jax.experimental import pallas as pl

DIM = 256
N_CODES = 8192
CHUNK = 2048
BR = 512


def _assign_kernel(x_ref, e2_ref, en_ref, ids_ref, out_ref):
    xb = x_ref[...]
    x_norm = jnp.sum(xb * xb, axis=1, keepdims=True)
    en = en_ref[...]
    e2 = e2_ref[...]

    best_idx = None
    run = None
    for c in range(N_CODES // CHUNK):
        mm2 = jax.lax.dot_general(
            xb, e2[:, c * CHUNK:(c + 1) * CHUNK],
            dimension_numbers=(((1,), (0,)), ((), ())),
            preferred_element_type=jnp.float32,
        )
        tc = (x_norm - mm2) + en[:, c * CHUNK:(c + 1) * CHUNK]
        m = jnp.min(tc, axis=1, keepdims=True)
        ids = ids_ref[:, c * CHUNK:(c + 1) * CHUNK]
        idx = jnp.min(jnp.where(tc == m, ids, 2.0 * N_CODES), axis=1)
        mval = m[:, 0]
        mval_b = mval.astype(jnp.bfloat16).astype(jnp.float32)
        if c == 0:
            best_idx = idx
            run = mval_b
        else:
            better = mval < run
            best_idx = jnp.where(better, idx, best_idx)
            run = jnp.where(better, mval_b, run)
    out_ref[0, 0, :] = best_idx.astype(jnp.int32)


@jax.jit
def kernel(x, embed):
    flat = x.reshape(-1, DIM)
    rows = flat.shape[0]
    nr = rows // BR
    embed2 = embed + embed
    embed_norm = jnp.sum(embed * embed, axis=0, keepdims=True)
    out = pl.pallas_call(
        _assign_kernel,
        grid=(nr,),
        in_specs=[
            pl.BlockSpec((BR, DIM), lambda i: (i, 0)),
            pl.BlockSpec((DIM, N_CODES), lambda i: (0, 0)),
            pl.BlockSpec((1, N_CODES), lambda i: (0, 0)),
            pl.BlockSpec((1, N_CODES), lambda i: (0, 0)),
        ],
        out_specs=pl.BlockSpec((1, 1, BR), lambda i: (i, 0, 0)),
        out_shape=jax.ShapeDtypeStruct((nr, 1, BR), jnp.int32),
    )(flat, embed2, embed_norm,
      jnp.arange(N_CODES, dtype=jnp.float32).reshape(1, N_CODES))
    return out.reshape(x.shape[:-1])
